# Initial kernel scaffold; baseline (speedup 1.0000x reference)
import jax
import jax.numpy as jnp
from jax.experimental import pallas as pl


def kernel(x, adj, A_tilde, adj_sct1, adj_sct2, adj_sct4, adj_sct8, adj_sct16, sct_index1, sct_index2, W0, W1, W2, W3, W4, b_gc1, W_res, b_res):
    raise NotImplementedError("write your pallas kernel here")


# speedup vs baseline: 2.1101x; 2.1101x over previous
import functools

import jax
import jax.numpy as jnp
from jax import lax
from jax.experimental import pallas as pl
from jax.experimental.pallas import tpu as pltpu

_N = 4096
_BM = 512
_SMOO = 0.1


def _feat_body(x_ref, w45_ref, w3_ref, w4_ref, t45_ref, t3_ref, t4_ref):
    x = x_ref[...]
    t45_ref[...] = jnp.dot(x, w45_ref[...], preferred_element_type=jnp.float32)
    t3_ref[...] = jnp.dot(x, w3_ref[...], preferred_element_type=jnp.float32)
    t4_ref[...] = jnp.dot(x, w4_ref[...], preferred_element_type=jnp.float32)


def _feat(x, w45, w3, w4):
    n, nfeat = x.shape
    return pl.pallas_call(
        _feat_body,
        grid=(n // _BM,),
        in_specs=[
            pl.BlockSpec((_BM, nfeat), lambda i: (i, 0)),
            pl.BlockSpec((nfeat, 45), lambda i: (0, 0)),
            pl.BlockSpec((nfeat, 30), lambda i: (0, 0)),
            pl.BlockSpec((nfeat, 30), lambda i: (0, 0)),
        ],
        out_specs=[
            pl.BlockSpec((_BM, 45), lambda i: (i, 0)),
            pl.BlockSpec((_BM, 30), lambda i: (i, 0)),
            pl.BlockSpec((_BM, 30), lambda i: (i, 0)),
        ],
        out_shape=[
            jax.ShapeDtypeStruct((n, 45), jnp.float32),
            jax.ShapeDtypeStruct((n, 30), jnp.float32),
            jax.ShapeDtypeStruct((n, 30), jnp.float32),
        ],
        compiler_params=pltpu.CompilerParams(
            dimension_semantics=("parallel",)),
    )(x, w45, w3, w4)


def _spmm_body(a_ref, b_ref, o_ref, *, take):
    o_ref[...] = jnp.dot(a_ref[...], b_ref[:, :take],
                         preferred_element_type=jnp.float32)


def _spmm(mat, rhs, take):
    n = mat.shape[0]
    wb = rhs.shape[1]
    return pl.pallas_call(
        functools.partial(_spmm_body, take=take),
        grid=(n // _BM,),
        in_specs=[
            pl.BlockSpec((_BM, n), lambda i: (i, 0)),
            pl.BlockSpec((n, wb), lambda i: (0, 0)),
        ],
        out_specs=pl.BlockSpec((_BM, take), lambda i: (i, 0)),
        out_shape=jax.ShapeDtypeStruct((n, take), jnp.float32),
        compiler_params=pltpu.CompilerParams(
            dimension_semantics=("parallel",)),
    )(mat, rhs)


def _support_body(a_ref, vfull_ref, u_ref, v_ref, p3_ref, p4_ref, bg_ref,
                  wr_ref, sup_ref):
    w = jnp.dot(a_ref[...], vfull_ref[:, :15],
                preferred_element_type=jnp.float32)
    h = jnp.concatenate(
        [u_ref[:, 30:45], v_ref[:, 15:30], w, p3_ref[...], p4_ref[...]],
        axis=1) + bg_ref[...]
    hh = h * h
    sup_ref[...] = jnp.dot(hh * hh, wr_ref[...],
                           preferred_element_type=jnp.float32)


def _support(a_tilde, v30, u45, p3, p4, bg, wres):
    n = a_tilde.shape[0]
    return pl.pallas_call(
        _support_body,
        grid=(n // _BM,),
        in_specs=[
            pl.BlockSpec((_BM, n), lambda i: (i, 0)),
            pl.BlockSpec((n, 30), lambda i: (0, 0)),
            pl.BlockSpec((_BM, 45), lambda i: (i, 0)),
            pl.BlockSpec((_BM, 30), lambda i: (i, 0)),
            pl.BlockSpec((_BM, 30), lambda i: (i, 0)),
            pl.BlockSpec((_BM, 30), lambda i: (i, 0)),
            pl.BlockSpec((1, 105), lambda i: (0, 0)),
            pl.BlockSpec((105, 16), lambda i: (0, 0)),
        ],
        out_specs=pl.BlockSpec((_BM, 16), lambda i: (i, 0)),
        out_shape=jax.ShapeDtypeStruct((n, 16), jnp.float32),
        compiler_params=pltpu.CompilerParams(
            dimension_semantics=("parallel",)),
    )(a_tilde, v30, u45, v30, p3, p4, bg, wres)


def _final_body(adj_ref, supk_ref, supm_ref, br_ref, out_ref):
    acc = jnp.dot(adj_ref[...], supk_ref[...],
                  preferred_element_type=jnp.float32)
    o = (_SMOO * acc + supm_ref[...]) / (1.0 + _SMOO) + br_ref[...]
    mx = jnp.max(o, axis=1, keepdims=True)
    shifted = o - mx
    out_ref[...] = shifted - jnp.log(
        jnp.sum(jnp.exp(shifted), axis=1, keepdims=True))


def _final(adj, support, br):
    n = adj.shape[0]
    return pl.pallas_call(
        _final_body,
        grid=(n // _BM,),
        in_specs=[
            pl.BlockSpec((_BM, n), lambda i: (i, 0)),
            pl.BlockSpec((n, 16), lambda i: (0, 0)),
            pl.BlockSpec((_BM, 16), lambda i: (i, 0)),
            pl.BlockSpec((1, 16), lambda i: (0, 0)),
        ],
        out_specs=pl.BlockSpec((_BM, 16), lambda i: (i, 0)),
        out_shape=jax.ShapeDtypeStruct((n, 16), jnp.float32),
        compiler_params=pltpu.CompilerParams(
            dimension_semantics=("parallel",)),
    )(adj, support, support, br)


def kernel(x, adj, A_tilde, adj_sct1, adj_sct2, adj_sct4, adj_sct8,
           adj_sct16, sct_index1, sct_index2, W0, W1, W2, W3, W4, b_gc1,
           W_res, b_res):
    w45 = jnp.concatenate([W2, W1, W0], axis=1)
    t45, t3, t4 = _feat(x, w45, W3, W4)

    u45 = _spmm(A_tilde, t45, 45)

    scat = (adj_sct1, adj_sct2, adj_sct4, adj_sct8, adj_sct16)
    i1 = jnp.asarray(sct_index1, dtype=jnp.int32)
    i2 = jnp.asarray(sct_index2, dtype=jnp.int32)
    p3 = lax.switch(i1, [lambda m=m: _spmm(m, t3, 30) for m in scat])
    p4 = lax.switch(i2, [lambda m=m: _spmm(m, t4, 30) for m in scat])

    v30 = _spmm(A_tilde, u45, 30)

    bg = b_gc1.reshape(1, 105)
    support = _support(A_tilde, v30, u45, p3, p4, bg, W_res)

    br = b_res.reshape(1, 16)
    return _final(adj, support, br)
